# Initial kernel scaffold; baseline (speedup 1.0000x reference)
import jax
import jax.numpy as jnp
from jax.experimental import pallas as pl


def kernel(x, edge_index, W_proj, b_proj, W1, W2):
    raise NotImplementedError("write your pallas kernel here")


# speedup vs baseline: 9.4703x; 9.4703x over previous
import functools

import jax
import jax.numpy as jnp
from jax import lax
from jax.experimental import pallas as pl
from jax.experimental.pallas import tpu as pltpu
from jax.experimental.pallas import tpu_sc as plsc

import numpy as np

_ALPHA = 0.5
_THETA = 1.0
_NC = 2
_NS = 16
_NW = _NC * _NS
_CHUNK = 128
_DEGW = 16


def _deg_kernel(n_acc, d, cpw):
    mesh = plsc.VectorSubcoreMesh(core_axis_name="c", subcore_axis_name="s")
    rpt = n_acc // _NS

    @functools.partial(
        pl.kernel,
        out_type=jax.ShapeDtypeStruct((_NC, n_acc, d), jnp.float32),
        mesh=mesh,
        scratch_types=[
            pltpu.VMEM((cpw, _CHUNK), jnp.int32),
            pltpu.VMEM((_CHUNK, d), jnp.float32),
            pltpu.VMEM_SHARED((n_acc, d), jnp.float32),
        ],
    )
    def kern(dst_hbm, zeros_hbm, ones_hbm, out_hbm, idx_d, ones_v, acc):
        c = lax.axis_index("c")
        s = lax.axis_index("s")
        wid = s * _NC + c
        pltpu.sync_copy(zeros_hbm.at[pl.ds(s * rpt, rpt)],
                        acc.at[pl.ds(s * rpt, rpt)])
        pltpu.sync_copy(ones_hbm, ones_v)
        pltpu.sync_copy(dst_hbm.at[wid], idx_d)
        plsc.subcore_barrier()

        def body(j, carry):
            pltpu.sync_copy(ones_v, acc.at[idx_d.at[j]], add=True)
            return carry

        lax.fori_loop(0, cpw, body, 0)
        plsc.subcore_barrier()
        pltpu.sync_copy(acc.at[pl.ds(s * rpt, rpt)],
                        out_hbm.at[c, pl.ds(s * rpt, rpt)])

    return kern


def _propagate_kernel(n_acc, d, cpw):
    mesh = plsc.VectorSubcoreMesh(core_axis_name="c", subcore_axis_name="s")
    rpt = n_acc // _NS

    @functools.partial(
        pl.kernel,
        out_type=jax.ShapeDtypeStruct((_NC, n_acc, d), jnp.float32),
        mesh=mesh,
        scratch_types=[
            pltpu.VMEM((cpw, _CHUNK), jnp.int32),
            pltpu.VMEM((cpw, _CHUNK), jnp.int32),
            pltpu.VMEM((_CHUNK, d), jnp.float32),
            pltpu.VMEM_SHARED((n_acc, d), jnp.float32),
            pltpu.SemaphoreType.DMA,
        ],
    )
    def kern(g_hbm, src_hbm, dst_hbm, zeros_hbm, out_hbm,
             idx_s, idx_d, rows, acc, sem):
        c = lax.axis_index("c")
        s = lax.axis_index("s")
        wid = s * _NC + c
        pltpu.sync_copy(zeros_hbm.at[pl.ds(s * rpt, rpt)],
                        acc.at[pl.ds(s * rpt, rpt)])
        pltpu.sync_copy(src_hbm.at[wid], idx_s)
        pltpu.sync_copy(dst_hbm.at[wid], idx_d)
        plsc.subcore_barrier()

        def body(j, carry):
            pltpu.async_copy(g_hbm.at[idx_s.at[j]], rows, sem).wait()
            pltpu.sync_copy(rows, acc.at[idx_d.at[j]], add=True)
            return carry

        lax.fori_loop(0, cpw, body, 0)
        plsc.subcore_barrier()
        pltpu.sync_copy(acc.at[pl.ds(s * rpt, rpt)],
                        out_hbm.at[c, pl.ds(s * rpt, rpt)])

    return kern


def _proj_tc(x, w, b2, degp, r):
    n, d = x.shape
    nb = n // r

    def body(x_ref, w_ref, b_ref, degp_ref, h0_ref, g0_ref):
        h0 = jnp.dot(x_ref[...], w_ref[...],
                     preferred_element_type=jnp.float32) + b_ref[...]
        dp = degp_ref[...]
        deg = dp[0, :, 0:1] + dp[1, :, 0:1] + 1.0
        dinv = lax.rsqrt(deg)
        h0_ref[...] = h0
        g0_ref[...] = dinv * h0

    return pl.pallas_call(
        body,
        grid=(nb,),
        in_specs=[
            pl.BlockSpec((r, d), lambda i: (i, 0)),
            pl.BlockSpec((d, d), lambda i: (0, 0)),
            pl.BlockSpec((1, d), lambda i: (0, 0)),
            pl.BlockSpec((_NC, r, d), lambda i: (0, i, 0)),
        ],
        out_specs=[
            pl.BlockSpec((r, d), lambda i: (i, 0)),
            pl.BlockSpec((r, d), lambda i: (i, 0)),
        ],
        out_shape=[
            jax.ShapeDtypeStruct((n, d), jnp.float32),
            jax.ShapeDtypeStruct((n, d), jnp.float32),
        ],
    )(x, w, b2, degp)


def _layer_tc(sp, degp, h, h0, w1, w2, beta, r):
    n, d = h.shape
    nb = n // r

    def body(sp_ref, degp_ref, h_ref, h0_ref, w1_ref, w2_ref, hn_ref, gn_ref):
        spv = sp_ref[...]
        sv = spv[0] + spv[1]
        dp = degp_ref[...]
        deg = dp[0, :, 0:1] + dp[1, :, 0:1] + 1.0
        dinv = lax.rsqrt(deg)
        agg = dinv * sv + (dinv * dinv) * h_ref[...]
        xh = (1.0 - _ALPHA) * agg
        x0a = _ALPHA * h0_ref[...]
        lin = (jnp.dot(xh, w1_ref[...], preferred_element_type=jnp.float32)
               + jnp.dot(x0a, w2_ref[...], preferred_element_type=jnp.float32))
        out = (1.0 - beta) * (xh + x0a) + beta * lin
        hn = jnp.maximum(out, 0.0)
        hn_ref[...] = hn
        gn_ref[...] = dinv * hn

    return pl.pallas_call(
        body,
        grid=(nb,),
        in_specs=[
            pl.BlockSpec((_NC, r, d), lambda i: (0, i, 0)),
            pl.BlockSpec((_NC, r, d), lambda i: (0, i, 0)),
            pl.BlockSpec((r, d), lambda i: (i, 0)),
            pl.BlockSpec((r, d), lambda i: (i, 0)),
            pl.BlockSpec((d, d), lambda i: (0, 0)),
            pl.BlockSpec((d, d), lambda i: (0, 0)),
        ],
        out_specs=[
            pl.BlockSpec((r, d), lambda i: (i, 0)),
            pl.BlockSpec((r, d), lambda i: (i, 0)),
        ],
        out_shape=[
            jax.ShapeDtypeStruct((n, d), jnp.float32),
            jax.ShapeDtypeStruct((n, d), jnp.float32),
        ],
    )(sp, degp, h, h0, w1, w2)


def kernel(x, edge_index, W_proj, b_proj, W1, W2):
    n, d = x.shape
    e = edge_index.shape[1]
    num_layers = W1.shape[0]

    assert n % _NS == 0 and d == 128

    src = edge_index[0].astype(jnp.int32)
    dst = edge_index[1].astype(jnp.int32)
    cpw = -(-e // (_NW * _CHUNK))
    e_pad = cpw * _NW * _CHUNK
    src = jnp.concatenate([src, jnp.zeros((e_pad - e,), jnp.int32)])
    dst = jnp.concatenate([dst, jnp.full((e_pad - e,), n, jnp.int32)])
    src3 = src.reshape(_NW, cpw, _CHUNK)
    dst3 = dst.reshape(_NW, cpw, _CHUNK)

    n_acc = ((n + 1 + _NS * 8 - 1) // (_NS * 8)) * (_NS * 8)
    zeros_d = jnp.zeros((n_acc, d), jnp.float32)
    ones_d = jnp.ones((_CHUNK, d), jnp.float32)
    b2 = b_proj.reshape(1, d)

    r = 1000
    assert n % r == 0

    degp = _deg_kernel(n_acc, d, cpw)(dst3, zeros_d, ones_d)

    h0, g = _proj_tc(x, W_proj, b2, degp, r)

    prop = _propagate_kernel(n_acc, d, cpw)
    h = h0
    for i in range(num_layers):
        beta = float(np.log(_THETA / (i + 1) + 1.0))
        sp = prop(g, src3, dst3, zeros_d)
        h, g = _layer_tc(sp, degp, h, h0, W1[i], W2[i], beta, r)
    return h
